# baseline (speedup 1.0000x reference)
import dataclasses
import functools

import jax
import jax.numpy as jnp
from jax import lax
from jax.experimental import pallas as pl
from jax.experimental.pallas import tpu as pltpu
from jax.experimental.pallas import tpu_sc as plsc

N = 65536
NNZ = 4294967

NC = 2
NS = 16
NW = NC * NS

CHUNK = 4096
CROWS = CHUNK // 128
STEPS = NNZ // (NW * CHUNK)
MAIN = STEPS * NW * CHUNK
PAIRS = STEPS // 2

LAST = NNZ - (NNZ % 8)
TAILSZ = LAST - MAIN
CT = -(-TAILSZ // NW)
WCAP = ((NNZ - CHUNK) // 8) * 8

PER_TILE = STEPS * CHUNK


def _make_sc_kernel():
    mesh = plsc.VectorSubcoreMesh(
        core_axis_name="c", subcore_axis_name="s", num_cores=NC, num_subcores=NS
    )

    cp = pltpu.CompilerParams()
    if "needs_layout_passes" in pltpu.CompilerParams.__dataclass_fields__:
        cp = dataclasses.replace(cp, needs_layout_passes=False)

    @functools.partial(
        pl.kernel,
        out_type=jax.ShapeDtypeStruct((NC, N), jnp.float32),
        mesh=mesh,
        compiler_params=cp,
        scratch_types=[
            pltpu.VMEM((N,), jnp.float32),
            [pltpu.VMEM((CHUNK,), jnp.float32)] * 2,
            [pltpu.VMEM((CHUNK,), jnp.int32)] * 2,
            [pltpu.VMEM((CHUNK,), jnp.int32)] * 2,
            [pltpu.VMEM((CHUNK,), jnp.float32)] * 2,
            pltpu.VMEM((16,), jnp.float32),
            pltpu.VMEM((16,), jnp.int32),
            pltpu.VMEM((16,), jnp.int32),
            pltpu.VMEM((16,), jnp.float32),
            pltpu.VMEM((N // NS,), jnp.float32),
            pltpu.VMEM_SHARED((N,), jnp.float32),
            [pltpu.SemaphoreType.DMA] * 2,
            [pltpu.SemaphoreType.DMA] * 2,
        ],
    )
    def sc_spmv(x_hbm, vals_hbm, cols_hbm, rows_hbm,
                lv_hbm, lc_hbm, lr_hbm,
                out_hbm,
                x_tile, vals_bufs, cols_bufs, rows_bufs, contrib_bufs,
                v16, c16, r16, k16,
                zbuf, acc, sems_in, sems_sc):
        cid = lax.axis_index("c")
        sid = lax.axis_index("s")
        wid = cid * NS + sid

        def fire_in(p, elem_base):
            elem_base = pl.multiple_of(elem_base, 8)
            pltpu.async_copy(vals_hbm.at[pl.ds(elem_base, CHUNK)],
                             vals_bufs[p], sems_in[p])
            pltpu.async_copy(cols_hbm.at[pl.ds(elem_base, CHUNK)],
                             cols_bufs[p], sems_in[p])
            pltpu.async_copy(rows_hbm.at[pl.ds(elem_base, CHUNK)],
                             rows_bufs[p], sems_in[p])

        def wait_in(p):
            pltpu.make_async_copy(vals_hbm.at[pl.ds(0, CHUNK)],
                                  vals_bufs[p], sems_in[p]).wait()
            pltpu.make_async_copy(cols_hbm.at[pl.ds(0, CHUNK)],
                                  cols_bufs[p], sems_in[p]).wait()
            pltpu.make_async_copy(rows_hbm.at[pl.ds(0, CHUNK)],
                                  rows_bufs[p], sems_in[p]).wait()

        def compute(p):
            vals_b, cols_b, contrib_b = vals_bufs[p], cols_bufs[p], contrib_bufs[p]

            @pl.loop(0, CHUNK, step=16, unroll=8)
            def _(i):
                cv = cols_b[pl.ds(i, 16)]
                xg = plsc.load_gather(x_tile, [cv])
                vv = vals_b[pl.ds(i, 16)]
                contrib_b[pl.ds(i, 16)] = vv * xg

        def compute_masked(p, wstart, lo, hi):
            vals_b, cols_b, contrib_b = vals_bufs[p], cols_bufs[p], contrib_bufs[p]
            lane = lax.iota(jnp.int32, 16)
            zero = jnp.zeros((16,), jnp.float32)

            @pl.loop(0, CHUNK, step=16, unroll=4)
            def _(i):
                g = (wstart + i) + lane
                m = (g >= lo) & (g < hi)
                cv = cols_b[pl.ds(i, 16)]
                xg = plsc.load_gather(x_tile, [cv])
                vv = vals_b[pl.ds(i, 16)]
                contrib_b[pl.ds(i, 16)] = jnp.where(m, vv * xg, zero)

        def fire_sc(p):
            return [pltpu.async_copy(
                contrib_bufs[p],
                acc.at[rows_bufs[p]],
                sems_sc[p], add=True)]

        def drain(ds):
            for d in ds:
                d.wait()

        zero16 = jnp.zeros((16,), jnp.float32)

        @pl.loop(0, N // NS, step=16)
        def _(i):
            zbuf[pl.ds(i, 16)] = zero16

        pltpu.sync_copy(zbuf, acc.at[pl.ds(sid * (N // NS), N // NS)])

        pltpu.sync_copy(x_hbm, x_tile)

        plsc.subcore_barrier()

        fire_in(0, wid * PER_TILE)
        fire_in(1, wid * PER_TILE + CHUNK)

        @pl.loop(0, PAIRS)
        def _(t):
            a = 2 * t
            wait_in(0)
            compute(0)
            ds0 = fire_sc(0)

            wait_in(1)
            compute(1)
            ds1 = fire_sc(1)

            drain(ds0)

            @pl.when(t < PAIRS - 1)
            def _():
                fire_in(0, wid * PER_TILE + (a + 2) * CHUNK)

            drain(ds1)

            @pl.when(t < PAIRS - 1)
            def _():
                fire_in(1, wid * PER_TILE + (a + 3) * CHUNK)

        lo = MAIN + wid * CT
        hi = jnp.minimum(lo + CT, LAST)
        wstart = jnp.minimum(lo - lax.rem(lo, 8), WCAP)
        fire_in(0, wstart)
        wait_in(0)
        compute_masked(0, wstart, lo, hi)
        ds0 = fire_sc(0)

        @pl.when(wid == 0)
        def _():
            pltpu.sync_copy(lv_hbm, v16)
            pltpu.sync_copy(lc_hbm, c16)
            pltpu.sync_copy(lr_hbm, r16)
            lane = lax.iota(jnp.int32, 16)
            m = lane >= (16 - (NNZ % 8))
            cv = c16[...]
            xg = plsc.load_gather(x_tile, [cv])
            k16[...] = jnp.where(m, v16[...] * xg,
                                 jnp.zeros((16,), jnp.float32))
            pltpu.async_copy(k16, acc.at[r16], sems_sc[1], add=True).wait()

        drain(ds0)

        plsc.subcore_barrier()

        sl = N // NS
        pltpu.sync_copy(acc.at[pl.ds(sid * sl, sl)],
                        out_hbm.at[cid, pl.ds(sid * sl, sl)])

    return sc_spmv


_sc_spmv = _make_sc_kernel()


def _tc_add_body(p_ref, o_ref):
    o_ref[...] = p_ref[0] + p_ref[1]


@jax.jit
def kernel(x, A_vals, A_rows, A_cols):
    lv = A_vals[NNZ - 16:]
    lc = A_cols[NNZ - 16:]
    lr = A_rows[NNZ - 16:]

    partials = _sc_spmv(x, A_vals, A_cols, A_rows, lv, lc, lr)

    out = pl.pallas_call(
        _tc_add_body,
        out_shape=jax.ShapeDtypeStruct((512, 128), jnp.float32),
    )(partials.reshape(NC, 512, 128))
    return out.reshape(N)


# speedup vs baseline: 1.2979x; 1.2979x over previous
import dataclasses
import functools

import jax
import jax.numpy as jnp
from jax import lax
from jax.experimental import pallas as pl
from jax.experimental.pallas import tpu as pltpu
from jax.experimental.pallas import tpu_sc as plsc

N = 65536
NNZ = 4294967

NC = 2
NS = 16
NW = NC * NS

CHUNK = 4096
CROWS = CHUNK // 128
STEPS = NNZ // (NW * CHUNK)
MAIN = STEPS * NW * CHUNK
PAIRS = STEPS // 2

LAST = NNZ - (NNZ % 8)
TAILSZ = LAST - MAIN
CT = -(-TAILSZ // NW)
WCAP = ((NNZ - CHUNK) // 8) * 8

PER_TILE = STEPS * CHUNK


def _make_sc_kernel():
    mesh = plsc.VectorSubcoreMesh(
        core_axis_name="c", subcore_axis_name="s", num_cores=NC, num_subcores=NS
    )

    cp = pltpu.CompilerParams()
    if "needs_layout_passes" in pltpu.CompilerParams.__dataclass_fields__:
        cp = dataclasses.replace(cp, needs_layout_passes=False)

    @functools.partial(
        pl.kernel,
        out_type=jax.ShapeDtypeStruct((NC, N), jnp.float32),
        mesh=mesh,
        compiler_params=cp,
        scratch_types=[
            pltpu.VMEM((N,), jnp.float32),
            [pltpu.VMEM((CHUNK,), jnp.float32)] * 2,
            [pltpu.VMEM((CHUNK,), jnp.int32)] * 2,
            [pltpu.VMEM((CHUNK,), jnp.int32)] * 2,
            [pltpu.VMEM((CHUNK,), jnp.float32)] * 2,
            pltpu.VMEM((16,), jnp.float32),
            pltpu.VMEM((16,), jnp.int32),
            pltpu.VMEM((16,), jnp.int32),
            pltpu.VMEM((16,), jnp.float32),
            pltpu.VMEM((N // NS,), jnp.float32),
            pltpu.VMEM_SHARED((N,), jnp.float32),
            [pltpu.SemaphoreType.DMA] * 2,
            [pltpu.SemaphoreType.DMA] * 2,
        ],
    )
    def sc_spmv(x_hbm, vals_hbm, cols_hbm, rows_hbm,
                lv_hbm, lc_hbm, lr_hbm,
                out_hbm,
                x_tile, vals_bufs, cols_bufs, rows_bufs, contrib_bufs,
                v16, c16, r16, k16,
                zbuf, acc, sems_in, sems_sc):
        cid = lax.axis_index("c")
        sid = lax.axis_index("s")
        wid = cid * NS + sid

        def fire_in(p, elem_base):
            elem_base = pl.multiple_of(elem_base, 8)
            pltpu.async_copy(vals_hbm.at[pl.ds(elem_base, CHUNK)],
                             vals_bufs[p], sems_in[p])
            pltpu.async_copy(cols_hbm.at[pl.ds(elem_base, CHUNK)],
                             cols_bufs[p], sems_in[p])
            pltpu.async_copy(rows_hbm.at[pl.ds(elem_base, CHUNK)],
                             rows_bufs[p], sems_in[p])

        def wait_in(p):
            pltpu.make_async_copy(vals_hbm.at[pl.ds(0, CHUNK)],
                                  vals_bufs[p], sems_in[p]).wait()
            pltpu.make_async_copy(cols_hbm.at[pl.ds(0, CHUNK)],
                                  cols_bufs[p], sems_in[p]).wait()
            pltpu.make_async_copy(rows_hbm.at[pl.ds(0, CHUNK)],
                                  rows_bufs[p], sems_in[p]).wait()

        def compute(p):
            vals_b, cols_b, contrib_b = vals_bufs[p], cols_bufs[p], contrib_bufs[p]

            @pl.loop(0, CROWS)
            def _(j):
                @pl.loop(0, 128, step=16, unroll=8)
                def _(c):
                    i = j * 128 + c
                    cv = cols_b[pl.ds(i, 16)]
                    xg = plsc.load_gather(x_tile, [cv])
                    vv = vals_b[pl.ds(i, 16)]
                    contrib_b[pl.ds(i, 16)] = vv * xg

        def compute_masked(p, wstart, lo, hi):
            vals_b, cols_b, contrib_b = vals_bufs[p], cols_bufs[p], contrib_bufs[p]
            lane = lax.iota(jnp.int32, 16)
            zero = jnp.zeros((16,), jnp.float32)

            @pl.loop(0, CROWS)
            def _(j):
                @pl.loop(0, 128, step=16, unroll=4)
                def _(c):
                    i = j * 128 + c
                    g = (wstart + i) + lane
                    m = (g >= lo) & (g < hi)
                    cv = cols_b[pl.ds(i, 16)]
                    xg = plsc.load_gather(x_tile, [cv])
                    vv = vals_b[pl.ds(i, 16)]
                    contrib_b[pl.ds(i, 16)] = jnp.where(m, vv * xg, zero)

        def fire_sc(p):
            return [pltpu.async_copy(
                contrib_bufs[p],
                acc.at[rows_bufs[p]],
                sems_sc[p], add=True)]

        def drain(ds):
            for d in ds:
                d.wait()

        zero16 = jnp.zeros((16,), jnp.float32)

        @pl.loop(0, N // NS, step=16)
        def _(i):
            zbuf[pl.ds(i, 16)] = zero16

        pltpu.sync_copy(zbuf, acc.at[pl.ds(sid * (N // NS), N // NS)])

        pltpu.sync_copy(x_hbm, x_tile)

        plsc.subcore_barrier()

        fire_in(0, wid * PER_TILE)
        fire_in(1, wid * PER_TILE + CHUNK)

        @pl.loop(0, PAIRS)
        def _(t):
            a = 2 * t
            wait_in(0)
            compute(0)
            ds0 = fire_sc(0)

            wait_in(1)
            compute(1)
            ds1 = fire_sc(1)

            drain(ds0)

            @pl.when(t < PAIRS - 1)
            def _():
                fire_in(0, wid * PER_TILE + (a + 2) * CHUNK)

            drain(ds1)

            @pl.when(t < PAIRS - 1)
            def _():
                fire_in(1, wid * PER_TILE + (a + 3) * CHUNK)

        lo = MAIN + wid * CT
        hi = jnp.minimum(lo + CT, LAST)
        wstart = jnp.minimum(lo - lax.rem(lo, 8), WCAP)
        fire_in(0, wstart)
        wait_in(0)
        compute_masked(0, wstart, lo, hi)
        ds0 = fire_sc(0)

        @pl.when(wid == 0)
        def _():
            pltpu.sync_copy(lv_hbm, v16)
            pltpu.sync_copy(lc_hbm, c16)
            pltpu.sync_copy(lr_hbm, r16)
            lane = lax.iota(jnp.int32, 16)
            m = lane >= (16 - (NNZ % 8))
            cv = c16[...]
            xg = plsc.load_gather(x_tile, [cv])
            k16[...] = jnp.where(m, v16[...] * xg,
                                 jnp.zeros((16,), jnp.float32))
            pltpu.async_copy(k16, acc.at[r16], sems_sc[1], add=True).wait()

        drain(ds0)

        plsc.subcore_barrier()

        sl = N // NS
        pltpu.sync_copy(acc.at[pl.ds(sid * sl, sl)],
                        out_hbm.at[cid, pl.ds(sid * sl, sl)])

    return sc_spmv


_sc_spmv = _make_sc_kernel()


def _tc_add_body(p_ref, o_ref):
    o_ref[...] = p_ref[0] + p_ref[1]


@jax.jit
def kernel(x, A_vals, A_rows, A_cols):
    lv = A_vals[NNZ - 16:]
    lc = A_cols[NNZ - 16:]
    lr = A_rows[NNZ - 16:]

    partials = _sc_spmv(x, A_vals, A_cols, A_rows, lv, lc, lr)

    out = pl.pallas_call(
        _tc_add_body,
        out_shape=jax.ShapeDtypeStruct((512, 128), jnp.float32),
    )(partials.reshape(NC, 512, 128))
    return out.reshape(N)
